# baseline (speedup 1.0000x reference)
import functools

import jax
import jax.numpy as jnp
from jax import lax
from jax.experimental import pallas as pl
from jax.experimental.pallas import tpu as pltpu


def _conv_kernel(x_ref, w_ref, b_ref, o_ref, *, K, L, GB):
    D = w_ref.shape[-1]
    taps = jnp.concatenate(
        [jnp.concatenate([x_ref[g, :, k:k + L] for k in range(K)], axis=0)
         for g in range(GB)], axis=1)
    acc = lax.dot_general(
        taps, w_ref[...],
        dimension_numbers=(((0,), (0,)), ((), ())),
        preferred_element_type=jnp.float32)
    acc = jnp.maximum(acc + b_ref[...], 0.0)
    o_ref[...] = acc.reshape(GB, L, D).astype(o_ref.dtype)


def kernel(x_ncl, w_kcd, b_row):
    B, C, L = x_ncl.shape
    K, _, D = w_kcd.shape
    pad = (K - 1) // 2
    halo = K - 1
    GB = 8 if B % 8 == 0 else 1
    xp = jnp.pad(x_ncl, ((0, 0), (0, 0), (pad, pad)))
    w_flat = w_kcd.reshape(K * C, D)
    body = functools.partial(_conv_kernel, K=K, L=L, GB=GB)
    return pl.pallas_call(
        body,
        out_shape=jax.ShapeDtypeStruct((B, L, D), jnp.float32),
        grid=(B // GB,),
        in_specs=[
            pl.BlockSpec((GB, C, L + halo), lambda b: (b, 0, 0)),
            pl.BlockSpec((K * C, D), lambda b: (0, 0)),
            pl.BlockSpec((1, D), lambda b: (0, 0)),
        ],
        out_specs=pl.BlockSpec((GB, L, D), lambda b: (b, 0, 0)),
        compiler_params=pltpu.CompilerParams(
            dimension_semantics=("parallel",),
            vmem_limit_bytes=63 * 1024 * 1024,
        ),
    )(xp, w_flat, b_row)


# speedup vs baseline: 1.0540x; 1.0540x over previous
import functools

import jax
import jax.numpy as jnp
from jax import lax
from jax.experimental import pallas as pl
from jax.experimental.pallas import tpu as pltpu


def _conv_kernel(x_ref, w_ref, b_ref, o_ref, *, K, L, GB):
    C = x_ref.shape[1]
    D = w_ref.shape[-1]
    pad = (K - 1) // 2

    def tap(row, k):
        if k < pad:
            return jnp.concatenate(
                [jnp.zeros((C, pad - k), jnp.float32), row[:, :L - (pad - k)]],
                axis=1)
        if k > pad:
            return jnp.concatenate(
                [row[:, k - pad:], jnp.zeros((C, k - pad), jnp.float32)],
                axis=1)
        return row

    taps = jnp.concatenate(
        [jnp.concatenate([tap(x_ref[g], k) for k in range(K)], axis=0)
         for g in range(GB)], axis=1)
    acc = lax.dot_general(
        taps, w_ref[...],
        dimension_numbers=(((0,), (0,)), ((), ())),
        preferred_element_type=jnp.float32)
    acc = jnp.maximum(acc + b_ref[...], 0.0)
    o_ref[...] = acc.reshape(GB, L, D).astype(o_ref.dtype)


def kernel(x_ncl, w_kcd, b_row):
    B, C, L = x_ncl.shape
    K, _, D = w_kcd.shape
    GB = 8 if B % 8 == 0 else 1
    w_flat = w_kcd.reshape(K * C, D)
    body = functools.partial(_conv_kernel, K=K, L=L, GB=GB)
    return pl.pallas_call(
        body,
        out_shape=jax.ShapeDtypeStruct((B, L, D), jnp.float32),
        grid=(B // GB,),
        in_specs=[
            pl.BlockSpec((GB, C, L), lambda b: (b, 0, 0)),
            pl.BlockSpec((K * C, D), lambda b: (0, 0)),
            pl.BlockSpec((1, D), lambda b: (0, 0)),
        ],
        out_specs=pl.BlockSpec((GB, L, D), lambda b: (b, 0, 0)),
        compiler_params=pltpu.CompilerParams(
            dimension_semantics=("parallel",),
            vmem_limit_bytes=63 * 1024 * 1024,
        ),
    )(x_ncl, w_flat, b_row)
